# Initial kernel scaffold; baseline (speedup 1.0000x reference)
import jax
import jax.numpy as jnp
from jax.experimental import pallas as pl


def kernel(q_mu, q_var):
    raise NotImplementedError("write your pallas kernel here")


# speedup vs baseline: 103.2821x; 103.2821x over previous
import functools

import jax
import jax.numpy as jnp
import numpy as np
from jax import lax
from jax.experimental import pallas as pl
from jax.experimental.pallas import tpu as pltpu
from jax.experimental.pallas import tpu_sc as plsc

N_OBJ = 50000
N_DIM = 256
ALPHA = 0.05
OBJ_BLK = 128
N_PAD = 50048
HB = 50176
HROWS = HB // 128
SENT_BIN = N_OBJ

NUM_WORKERS = 32
DIMS_PER_WORKER = N_DIM // NUM_WORKERS
VECS = N_PAD // 16
HVECS = HB // 16
UNROLL = 4


def _stage_a_body(mu_ref, var_ref, out_ref):
  i = pl.program_id(0)
  mu = mu_ref[...]
  var = var_ref[...]
  p = 0.5 * (1.0 + lax.erf((0.0 - mu) / (var * np.float32(np.sqrt(np.float32(2.0))))))
  a = np.float32(ALPHA)
  nf = np.float32(N_OBJ)
  q = p / a
  m = q * nf
  k0 = jnp.clip(jnp.ceil(m), 3.0, nf - 2.0)
  s = jnp.zeros_like(p)
  for j in range(-2, 3):
    t = ((k0 + np.float32(j)) / nf) * a
    s = s + jnp.where(t >= p, 1.0, 0.0)
  r = k0 + 3.0 - s
  r = jnp.clip(r, 1.0, nf + 1.0)
  binv = r - 1.0
  obj = lax.broadcasted_iota(jnp.int32, (OBJ_BLK, N_DIM), 0) + i * OBJ_BLK
  binv = jnp.where(obj < N_OBJ, binv, np.float32(SENT_BIN))
  out_ref[...] = jnp.transpose(binv).astype(jnp.int32)


def _stage_a(q_mu, q_var):
  return pl.pallas_call(
      _stage_a_body,
      grid=(N_PAD // OBJ_BLK,),
      in_specs=[
          pl.BlockSpec((OBJ_BLK, N_DIM), lambda i: (i, 0)),
          pl.BlockSpec((OBJ_BLK, N_DIM), lambda i: (i, 0)),
      ],
      out_specs=pl.BlockSpec((N_DIM, OBJ_BLK), lambda i: (0, i)),
      out_shape=jax.ShapeDtypeStruct((N_DIM, N_PAD), jnp.int32),
  )(q_mu, q_var)


def _sc_hist_body(r_hbm, hist_hbm, row_v, hist_v):
  wid = lax.axis_index("s") * 2 + lax.axis_index("c")

  for d in range(DIMS_PER_WORKER):
    dim = wid * DIMS_PER_WORKER + d
    pltpu.sync_copy(r_hbm.at[dim], row_v)

    def _zero(z, carry):
      for u in range(UNROLL):
        hist_v[pl.ds((z * UNROLL + u) * 16, 16)] = jnp.zeros((16,), jnp.int32)
      return carry
    lax.fori_loop(0, HVECS // UNROLL, _zero, 0)

    def _scatter(it, carry):
      for u in range(UNROLL):
        v = row_v[pl.ds((it * UNROLL + u) * 16, 16)]
        plsc.addupdate_scatter(hist_v, [v], jnp.ones((16,), jnp.int32))
      return carry
    lax.fori_loop(0, VECS // UNROLL, _scatter, 0)

    pltpu.sync_copy(hist_v, hist_hbm.at[dim])


def _sc_hist(r_t):
  mesh = plsc.VectorSubcoreMesh(
      core_axis_name="c", subcore_axis_name="s", num_cores=2, num_subcores=16)
  f = pl.kernel(
      _sc_hist_body,
      out_type=jax.ShapeDtypeStruct((N_DIM, HB), jnp.int32),
      mesh=mesh,
      compiler_params=pltpu.CompilerParams(needs_layout_passes=False),
      scratch_types=[
          pltpu.VMEM((N_PAD,), jnp.int32),
          pltpu.VMEM((HB,), jnp.int32),
      ],
  )
  return f(r_t)


DIM_BLK = 32


def _stage_c_body(hist_ref, out_ref):
  h = hist_ref[...].astype(jnp.float32).reshape(DIM_BLK, HROWS, 128)
  ia = lax.broadcasted_iota(jnp.int32, (128, 128), 0)
  ib = lax.broadcasted_iota(jnp.int32, (128, 128), 1)
  tri128 = jnp.where(ia <= ib, 1.0, 0.0)
  ra = lax.broadcasted_iota(jnp.int32, (HROWS, HROWS), 0)
  rb = lax.broadcasted_iota(jnp.int32, (HROWS, HROWS), 1)
  triR = jnp.where(ra < rb, 1.0, 0.0)

  h2 = h.reshape(DIM_BLK * HROWS, 128)
  cum = lax.dot_general(h2, tri128, (((1,), (0,)), ((), ())),
                        precision=lax.Precision.HIGHEST,
                        preferred_element_type=jnp.float32)
  cum = cum.reshape(DIM_BLK, HROWS, 128)
  rowsum = cum[:, :, 127]
  offs = lax.dot_general(rowsum, triR, (((1,), (0,)), ((), ())),
                         precision=lax.Precision.HIGHEST,
                         preferred_element_type=jnp.float32)
  g = cum + offs[:, :, None]

  kf = (lax.broadcasted_iota(jnp.int32, (HROWS, 128), 0) * 128
        + lax.broadcasted_iota(jnp.int32, (HROWS, 128), 1) + 1
        ).astype(jnp.float32)
  cond = (g >= kf[None]) & (kf[None] <= np.float32(N_OBJ))
  cnt = jnp.sum(jnp.sum(jnp.where(cond, 1.0, 0.0), axis=2), axis=1,
                keepdims=True)
  first = jnp.min(jnp.min(jnp.where(cond, kf[None], np.float32(1e9)), axis=2),
                  axis=1, keepdims=True)
  out_ref[...] = jnp.where(cnt > 0.0, cnt + first - 1.0, 0.0)


def _stage_c(hist):
  return pl.pallas_call(
      _stage_c_body,
      grid=(N_DIM // DIM_BLK,),
      in_specs=[pl.BlockSpec((DIM_BLK, HB), lambda i: (i, 0))],
      out_specs=pl.BlockSpec((DIM_BLK, 1), lambda i: (i, 0)),
      out_shape=jax.ShapeDtypeStruct((N_DIM, 1), jnp.float32),
  )(hist)


def kernel(q_mu, q_var):
  r_t = _stage_a(q_mu, q_var)
  hist = _sc_hist(r_t)
  imp = _stage_c(hist)
  return imp.reshape(N_DIM)


# speedup vs baseline: 120.3231x; 1.1650x over previous
import functools

import jax
import jax.numpy as jnp
import numpy as np
from jax import lax
from jax.experimental import pallas as pl
from jax.experimental.pallas import tpu as pltpu
from jax.experimental.pallas import tpu_sc as plsc

N_OBJ = 50000
N_DIM = 256
ALPHA = 0.05
OBJ_BLK = 128
N_PAD = 50048
HB = 50176
HROWS = HB // 128
SENT_BIN = N_OBJ

NUM_WORKERS = 32
DIMS_PER_WORKER = N_DIM // NUM_WORKERS
VECS = N_PAD // 16
HVECS = HB // 16
UNROLL = 8


def _stage_a_body(mu_ref, var_ref, out_ref):
  i = pl.program_id(0)
  mu = mu_ref[...]
  var = var_ref[...]
  p = 0.5 * (1.0 + lax.erf((0.0 - mu) / (var * np.float32(np.sqrt(np.float32(2.0))))))
  a = np.float32(ALPHA)
  nf = np.float32(N_OBJ)
  q = p / a
  m = q * nf
  k0 = jnp.clip(jnp.ceil(m), 3.0, nf - 2.0)
  s = jnp.zeros_like(p)
  for j in range(-2, 3):
    t = ((k0 + np.float32(j)) / nf) * a
    s = s + jnp.where(t >= p, 1.0, 0.0)
  r = k0 + 3.0 - s
  r = jnp.clip(r, 1.0, nf + 1.0)
  binv = r - 1.0
  obj = lax.broadcasted_iota(jnp.int32, (OBJ_BLK, N_DIM), 0) + i * OBJ_BLK
  binv = jnp.where(obj < N_OBJ, binv, np.float32(SENT_BIN))
  out_ref[...] = jnp.transpose(binv).astype(jnp.int32)


def _stage_a(q_mu, q_var):
  return pl.pallas_call(
      _stage_a_body,
      grid=(N_PAD // OBJ_BLK,),
      in_specs=[
          pl.BlockSpec((OBJ_BLK, N_DIM), lambda i: (i, 0)),
          pl.BlockSpec((OBJ_BLK, N_DIM), lambda i: (i, 0)),
      ],
      out_specs=pl.BlockSpec((N_DIM, OBJ_BLK), lambda i: (0, i)),
      out_shape=jax.ShapeDtypeStruct((N_DIM, N_PAD), jnp.int32),
  )(q_mu, q_var)


def _sc_hist_body(r_hbm, hist_hbm, row_v, hist_v):
  wid = lax.axis_index("s") * 2 + lax.axis_index("c")

  for d in range(DIMS_PER_WORKER):
    dim = wid * DIMS_PER_WORKER + d
    pltpu.sync_copy(r_hbm.at[dim], row_v)

    def _zero(z, carry):
      for u in range(UNROLL):
        hist_v[pl.ds((z * UNROLL + u) * 16, 16)] = jnp.zeros((16,), jnp.int32)
      return carry
    lax.fori_loop(0, HVECS // UNROLL, _zero, 0)

    def _scatter(it, carry):
      for u in range(UNROLL):
        v = row_v[pl.ds((it * UNROLL + u) * 16, 16)]
        plsc.addupdate_scatter(hist_v, [v], jnp.ones((16,), jnp.int32),
                               mask=v < SENT_BIN)
      return carry
    lax.fori_loop(0, VECS // UNROLL, _scatter, 0)

    pltpu.sync_copy(hist_v, hist_hbm.at[dim])


def _sc_hist(r_t):
  mesh = plsc.VectorSubcoreMesh(
      core_axis_name="c", subcore_axis_name="s", num_cores=2, num_subcores=16)
  f = pl.kernel(
      _sc_hist_body,
      out_type=jax.ShapeDtypeStruct((N_DIM, HB), jnp.int32),
      mesh=mesh,
      compiler_params=pltpu.CompilerParams(needs_layout_passes=False),
      scratch_types=[
          pltpu.VMEM((N_PAD,), jnp.int32),
          pltpu.VMEM((HB,), jnp.int32),
      ],
  )
  return f(r_t)


DIM_BLK = 32


def _stage_c_body(hist_ref, out_ref):
  h = hist_ref[...].astype(jnp.float32).reshape(DIM_BLK, HROWS, 128)
  ia = lax.broadcasted_iota(jnp.int32, (128, 128), 0)
  ib = lax.broadcasted_iota(jnp.int32, (128, 128), 1)
  tri128 = jnp.where(ia <= ib, 1.0, 0.0)
  ra = lax.broadcasted_iota(jnp.int32, (HROWS, HROWS), 0)
  rb = lax.broadcasted_iota(jnp.int32, (HROWS, HROWS), 1)
  triR = jnp.where(ra < rb, 1.0, 0.0)

  h2 = h.reshape(DIM_BLK * HROWS, 128)
  cum = lax.dot_general(h2, tri128, (((1,), (0,)), ((), ())),
                        precision=lax.Precision.HIGHEST,
                        preferred_element_type=jnp.float32)
  cum = cum.reshape(DIM_BLK, HROWS, 128)
  rowsum = cum[:, :, 127]
  offs = lax.dot_general(rowsum, triR, (((1,), (0,)), ((), ())),
                         precision=lax.Precision.HIGHEST,
                         preferred_element_type=jnp.float32)
  g = cum + offs[:, :, None]

  kf = (lax.broadcasted_iota(jnp.int32, (HROWS, 128), 0) * 128
        + lax.broadcasted_iota(jnp.int32, (HROWS, 128), 1) + 1
        ).astype(jnp.float32)
  cond = (g >= kf[None]) & (kf[None] <= np.float32(N_OBJ))
  cnt = jnp.sum(jnp.sum(jnp.where(cond, 1.0, 0.0), axis=2), axis=1,
                keepdims=True)
  first = jnp.min(jnp.min(jnp.where(cond, kf[None], np.float32(1e9)), axis=2),
                  axis=1, keepdims=True)
  out_ref[...] = jnp.where(cnt > 0.0, cnt + first - 1.0, 0.0)


def _stage_c(hist):
  return pl.pallas_call(
      _stage_c_body,
      grid=(N_DIM // DIM_BLK,),
      in_specs=[pl.BlockSpec((DIM_BLK, HB), lambda i: (i, 0))],
      out_specs=pl.BlockSpec((DIM_BLK, 1), lambda i: (i, 0)),
      out_shape=jax.ShapeDtypeStruct((N_DIM, 1), jnp.float32),
  )(hist)


def kernel(q_mu, q_var):
  r_t = _stage_a(q_mu, q_var)
  hist = _sc_hist(r_t)
  imp = _stage_c(hist)
  return imp.reshape(N_DIM)
